# baseline (speedup 1.0000x reference)
import functools

import jax
import jax.numpy as jnp
from jax import lax
from jax.experimental import pallas as pl
from jax.experimental.pallas import tpu as pltpu
from jax.experimental.pallas import tpu_sc as plsc

_BS = 4
_MAX_SEQLEN = 128
_LAYERS = 6
_HEADS = 32
_HEAD_DIM = 128
_STRIDE = 16
_NUM_PAGES = 64
_FEAT = _HEADS * _HEAD_DIM
_RPP = _LAYERS * 2 * _STRIDE
_N_ROWS = _NUM_PAGES * _RPP
_N_MEMBER = _BS * (_MAX_SEQLEN // _STRIDE)
_N_VALS = _LAYERS * 2
_CHUNK = 8
_FCHUNK = 4
_NW = 32
_NFILLW = 2 * _N_VALS
_NC_FILL = 160 // _CHUNK
_NC_COPY = 288 // _CHUNK
_NF = 256 // _FCHUNK


def _layer_consts():
    x = jnp.float32(1.0)
    ks, vs = [], []
    for _ in range(_LAYERS):
        xk = x * jnp.float32(2.0)
        xv = x * jnp.float32(4.0)
        ks.append(xk)
        vs.append(xv)
        x = x + x * xk * xv
    return ks, vs, x


def _copy_pipeline(nc, in_hbm, out_hbm, cidx_v, bufs, gsems, ssems):
    pltpu.make_async_copy(in_hbm.at[cidx_v.at[0]], bufs[0], gsems[0]).start()

    def step(j, carry):
        for r in range(3):
            @pl.when(j % 3 == r)
            def _(r=r):
                rn = (r + 1) % 3
                pltpu.make_async_copy(in_hbm.at[cidx_v.at[j]], bufs[r],
                                      gsems[r]).wait()
                pltpu.make_async_copy(bufs[r], out_hbm.at[cidx_v.at[j]],
                                      ssems[r]).start()

                @pl.when(j + 1 < nc)
                def _():
                    @pl.when(j >= 2)
                    def _():
                        pltpu.make_async_copy(
                            bufs[rn], out_hbm.at[cidx_v.at[0]],
                            ssems[rn]).wait()
                    pltpu.make_async_copy(in_hbm.at[cidx_v.at[j + 1]],
                                          bufs[rn], gsems[rn]).start()
        return carry

    lax.fori_loop(0, nc, step, 0)
    for jj in (nc - 2, nc - 1):
        pltpu.make_async_copy(bufs[jj % 3], out_hbm.at[cidx_v.at[0]],
                              ssems[jj % 3]).wait()


_MESH = plsc.VectorSubcoreMesh(core_axis_name="c", subcore_axis_name="s")


@functools.partial(
    pl.kernel,
    out_type=jax.ShapeDtypeStruct((_N_ROWS, _HEADS, _HEAD_DIM), jnp.float32),
    mesh=_MESH,
    scratch_types=[
        pltpu.VMEM((_NC_COPY, _CHUNK), jnp.int32),
        pltpu.VMEM((_NF, _FCHUNK), jnp.int32),
        pltpu.VMEM((_CHUNK, _HEADS, _HEAD_DIM), jnp.float32),
        pltpu.VMEM((_CHUNK, _HEADS, _HEAD_DIM), jnp.float32),
        pltpu.VMEM((_CHUNK, _HEADS, _HEAD_DIM), jnp.float32),
        pltpu.VMEM((_FCHUNK, _HEADS, _HEAD_DIM), jnp.float32),
        pltpu.SemaphoreType.DMA,
        pltpu.SemaphoreType.DMA,
        pltpu.SemaphoreType.DMA,
        pltpu.SemaphoreType.DMA,
        pltpu.SemaphoreType.DMA,
        pltpu.SemaphoreType.DMA,
        pltpu.SemaphoreType.DMA,
        pltpu.SemaphoreType.DMA,
    ],
)
def _sc_slab(in_hbm, cidx_hbm, fidx_hbm, out_hbm,
             cidx_v, fidx_v, buf0, buf1, buf2, fbuf,
             gsem0, gsem1, gsem2, ssem0, ssem1, ssem2, fsem, isem):
    c = lax.axis_index("c")
    s = lax.axis_index("s")
    w = s * 2 + c
    is_fill = w < _NFILLW
    bufs = (buf0, buf1, buf2)
    gsems = (gsem0, gsem1, gsem2)
    ssems = (ssem0, ssem1, ssem2)

    pltpu.make_async_copy(cidx_hbm.at[w], cidx_v, isem).start()
    pltpu.make_async_copy(fidx_hbm.at[w], fidx_v, isem).start()
    pltpu.make_async_copy(cidx_hbm.at[w], cidx_v, isem).wait()
    pltpu.make_async_copy(fidx_hbm.at[w], fidx_v, isem).wait()

    ks, vs, _ = _layer_consts()
    v = w // 2
    vec = jnp.zeros((16,), jnp.float32)
    for i in range(_N_VALS):
        cval = ks[i // 2] if i % 2 == 0 else vs[i // 2]
        vec = jnp.where(v == i, jnp.full((16,), cval), vec)

    @pl.when(is_fill)
    def _():
        def fill_row(r, carry):
            for k in range(_HEADS):
                row = fbuf.at[r, k]
                for m in range(_HEAD_DIM // 16):
                    row[pl.ds(m * 16, 16)] = vec
            return carry

        lax.fori_loop(0, _FCHUNK, fill_row, 0)

        def fill_start(j, carry):
            pltpu.make_async_copy(fbuf, out_hbm.at[fidx_v.at[j]],
                                  fsem).start()
            return carry

        lax.fori_loop(0, _NF, fill_start, 0)

        _copy_pipeline(_NC_FILL, in_hbm, out_hbm, cidx_v, bufs, gsems, ssems)

        def fill_drain(j, carry):
            pltpu.make_async_copy(fbuf, out_hbm.at[fidx_v.at[0]],
                                  fsem).wait()
            return carry

        lax.fori_loop(0, _NF, fill_drain, 0)

    @pl.when(jnp.logical_not(is_fill))
    def _():
        _copy_pipeline(_NC_COPY, in_hbm, out_hbm, cidx_v, bufs, gsems, ssems)


def _h_body(h_ref):
    _, _, h_final = _layer_consts()
    h_ref[...] = jnp.full((_BS, _MAX_SEQLEN, _FEAT), h_final)


def kernel(seq_lens, attn_block_ids, attn_page_slab):
    del seq_lens
    ids = attn_block_ids.reshape(-1).astype(jnp.int32)
    mask = jnp.zeros((_NUM_PAGES,), jnp.int32).at[ids].set(1)
    nm_pages = jnp.argsort(mask, stable=True)[: _NUM_PAGES - _N_MEMBER]
    nm_pages = nm_pages.astype(jnp.int32)

    copy_rows = (nm_pages[:, None] * _RPP
                 + jnp.arange(_RPP, dtype=jnp.int32)[None, :]).reshape(-1)
    n1 = _NFILLW * _NC_FILL * _CHUNK
    c1 = jnp.pad(copy_rows[:n1].reshape(_NFILLW, n1 // _NFILLW),
                 ((0, 0), (0, (_NC_COPY - _NC_FILL) * _CHUNK)))
    c2 = copy_rows[n1:].reshape(_NW - _NFILLW, _NC_COPY * _CHUNK)
    copy_idx = jnp.concatenate([c1, c2], axis=0).reshape(
        _NW, _NC_COPY, _CHUNK)

    fill_rows = (ids[None, :, None] * _RPP
                 + (jnp.arange(_N_VALS, dtype=jnp.int32) * _STRIDE)[:, None, None]
                 + jnp.arange(_STRIDE, dtype=jnp.int32)[None, None, :])
    fill_rows = fill_rows.reshape(_NFILLW, _NF * _FCHUNK)
    fill_idx = jnp.pad(fill_rows, ((0, _NW - _NFILLW), (0, 0)))
    fill_idx = fill_idx.reshape(_NW, _NF, _FCHUNK)

    slab_rows = attn_page_slab.reshape(_N_ROWS, _HEADS, _HEAD_DIM)
    out_rows = _sc_slab(slab_rows, copy_idx, fill_idx)
    slab_out = out_rows.reshape(
        _NUM_PAGES, _LAYERS, 2, _STRIDE, _HEADS, _HEAD_DIM)

    h = pl.pallas_call(
        _h_body,
        out_shape=jax.ShapeDtypeStruct((_BS, _MAX_SEQLEN, _FEAT), jnp.float32),
    )()
    return h, slab_out


# speedup vs baseline: 1.0282x; 1.0282x over previous
import functools

import jax
import jax.numpy as jnp
from jax import lax
from jax.experimental import pallas as pl
from jax.experimental.pallas import tpu as pltpu
from jax.experimental.pallas import tpu_sc as plsc

_BS = 4
_MAX_SEQLEN = 128
_LAYERS = 6
_HEADS = 32
_HEAD_DIM = 128
_STRIDE = 16
_NUM_PAGES = 64
_FEAT = _HEADS * _HEAD_DIM
_RPP = _LAYERS * 2 * _STRIDE
_N_ROWS = _NUM_PAGES * _RPP
_N_MEMBER = _BS * (_MAX_SEQLEN // _STRIDE)
_N_VALS = _LAYERS * 2
_CHUNK = 8
_FCHUNK = 4
_NW = 32
_NFILLW = 2 * _N_VALS
_NC_FILL = 152 // _CHUNK
_NC_COPY = 312 // _CHUNK
_NF = 256 // _FCHUNK


def _layer_consts():
    x = jnp.float32(1.0)
    ks, vs = [], []
    for _ in range(_LAYERS):
        xk = x * jnp.float32(2.0)
        xv = x * jnp.float32(4.0)
        ks.append(xk)
        vs.append(xv)
        x = x + x * xk * xv
    return ks, vs, x


def _copy_pipeline(nc, in_hbm, out_hbm, cidx_v, bufs, gsems, ssems):
    pltpu.make_async_copy(in_hbm.at[cidx_v.at[0]], bufs[0], gsems[0]).start()

    def step(j, carry):
        for r in range(3):
            @pl.when(j % 3 == r)
            def _(r=r):
                rn = (r + 1) % 3
                pltpu.make_async_copy(in_hbm.at[cidx_v.at[j]], bufs[r],
                                      gsems[r]).wait()
                pltpu.make_async_copy(bufs[r], out_hbm.at[cidx_v.at[j]],
                                      ssems[r]).start()

                @pl.when(j + 1 < nc)
                def _():
                    @pl.when(j >= 2)
                    def _():
                        pltpu.make_async_copy(
                            bufs[rn], out_hbm.at[cidx_v.at[0]],
                            ssems[rn]).wait()
                    pltpu.make_async_copy(in_hbm.at[cidx_v.at[j + 1]],
                                          bufs[rn], gsems[rn]).start()
        return carry

    lax.fori_loop(0, nc, step, 0)
    for jj in (nc - 2, nc - 1):
        pltpu.make_async_copy(bufs[jj % 3], out_hbm.at[cidx_v.at[0]],
                              ssems[jj % 3]).wait()


_MESH = plsc.VectorSubcoreMesh(core_axis_name="c", subcore_axis_name="s")


@functools.partial(
    pl.kernel,
    out_type=jax.ShapeDtypeStruct((_N_ROWS, _HEADS, _HEAD_DIM), jnp.float32),
    mesh=_MESH,
    scratch_types=[
        pltpu.VMEM((_NC_COPY, _CHUNK), jnp.int32),
        pltpu.VMEM((_NF, _FCHUNK), jnp.int32),
        pltpu.VMEM((_CHUNK, _HEADS, _HEAD_DIM), jnp.float32),
        pltpu.VMEM((_CHUNK, _HEADS, _HEAD_DIM), jnp.float32),
        pltpu.VMEM((_CHUNK, _HEADS, _HEAD_DIM), jnp.float32),
        pltpu.VMEM((_FCHUNK, _HEADS, _HEAD_DIM), jnp.float32),
        pltpu.SemaphoreType.DMA,
        pltpu.SemaphoreType.DMA,
        pltpu.SemaphoreType.DMA,
        pltpu.SemaphoreType.DMA,
        pltpu.SemaphoreType.DMA,
        pltpu.SemaphoreType.DMA,
        pltpu.SemaphoreType.DMA,
        pltpu.SemaphoreType.DMA,
    ],
)
def _sc_slab(in_hbm, cidx_hbm, fidx_hbm, out_hbm,
             cidx_v, fidx_v, buf0, buf1, buf2, fbuf,
             gsem0, gsem1, gsem2, ssem0, ssem1, ssem2, fsem, isem):
    c = lax.axis_index("c")
    s = lax.axis_index("s")
    w = s * 2 + c
    is_fill = w < _NFILLW
    bufs = (buf0, buf1, buf2)
    gsems = (gsem0, gsem1, gsem2)
    ssems = (ssem0, ssem1, ssem2)

    pltpu.make_async_copy(cidx_hbm.at[w], cidx_v, isem).start()
    pltpu.make_async_copy(fidx_hbm.at[w], fidx_v, isem).start()

    ks, vs, _ = _layer_consts()
    v = w // 2
    vec = jnp.zeros((16,), jnp.float32)
    for i in range(_N_VALS):
        cval = ks[i // 2] if i % 2 == 0 else vs[i // 2]
        vec = jnp.where(v == i, jnp.full((16,), cval), vec)

    @pl.when(is_fill)
    def _():
        def fill_row(r, carry):
            for k in range(_HEADS):
                row = fbuf.at[r, k]
                for m in range(_HEAD_DIM // 16):
                    row[pl.ds(m * 16, 16)] = vec
            return carry

        lax.fori_loop(0, _FCHUNK, fill_row, 0)

    pltpu.make_async_copy(cidx_hbm.at[w], cidx_v, isem).wait()
    pltpu.make_async_copy(fidx_hbm.at[w], fidx_v, isem).wait()

    @pl.when(is_fill)
    def _():
        def fill_start(j, carry):
            pltpu.make_async_copy(fbuf, out_hbm.at[fidx_v.at[j]],
                                  fsem).start()
            return carry

        lax.fori_loop(0, _NF, fill_start, 0)

        _copy_pipeline(_NC_FILL, in_hbm, out_hbm, cidx_v, bufs, gsems, ssems)

        def fill_drain(j, carry):
            pltpu.make_async_copy(fbuf, out_hbm.at[fidx_v.at[0]],
                                  fsem).wait()
            return carry

        lax.fori_loop(0, _NF, fill_drain, 0)

    @pl.when(jnp.logical_not(is_fill))
    def _():
        _copy_pipeline(_NC_COPY, in_hbm, out_hbm, cidx_v, bufs, gsems, ssems)


def _h_body(h_ref):
    _, _, h_final = _layer_consts()
    h_ref[...] = jnp.full((_BS, _MAX_SEQLEN, _FEAT), h_final)


def kernel(seq_lens, attn_block_ids, attn_page_slab):
    del seq_lens
    ids = attn_block_ids.reshape(-1).astype(jnp.int32)
    mask = jnp.zeros((_NUM_PAGES,), jnp.int32).at[ids].set(1)
    nm_pages = jnp.argsort(mask, stable=True)[: _NUM_PAGES - _N_MEMBER]
    nm_pages = nm_pages.astype(jnp.int32)

    copy_rows = (nm_pages[:, None] * _RPP
                 + jnp.arange(_RPP, dtype=jnp.int32)[None, :]).reshape(-1)
    n1 = _NFILLW * _NC_FILL * _CHUNK
    c1 = jnp.pad(copy_rows[:n1].reshape(_NFILLW, n1 // _NFILLW),
                 ((0, 0), (0, (_NC_COPY - _NC_FILL) * _CHUNK)))
    c2 = copy_rows[n1:].reshape(_NW - _NFILLW, _NC_COPY * _CHUNK)
    copy_idx = jnp.concatenate([c1, c2], axis=0).reshape(
        _NW, _NC_COPY, _CHUNK)

    fill_rows = (ids[None, :, None] * _RPP
                 + (jnp.arange(_N_VALS, dtype=jnp.int32) * _STRIDE)[:, None, None]
                 + jnp.arange(_STRIDE, dtype=jnp.int32)[None, None, :])
    fill_rows = fill_rows.reshape(_NFILLW, _NF * _FCHUNK)
    fill_idx = jnp.pad(fill_rows, ((0, _NW - _NFILLW), (0, 0)))
    fill_idx = fill_idx.reshape(_NW, _NF, _FCHUNK)

    slab_rows = attn_page_slab.reshape(_N_ROWS, _HEADS, _HEAD_DIM)
    out_rows = _sc_slab(slab_rows, copy_idx, fill_idx)
    slab_out = out_rows.reshape(
        _NUM_PAGES, _LAYERS, 2, _STRIDE, _HEADS, _HEAD_DIM)

    h = pl.pallas_call(
        _h_body,
        out_shape=jax.ShapeDtypeStruct((_BS, _MAX_SEQLEN, _FEAT), jnp.float32),
    )()
    return h, slab_out
